# baseline (speedup 1.0000x reference)
import functools

import jax
import jax.numpy as jnp
from jax import lax
from jax.experimental import pallas as pl
from jax.experimental.pallas import tpu as pltpu
from jax.experimental.pallas import tpu_sc as plsc

_EPS = 1e-6
_MARGIN = 1.0


def _sc_gather(U_true, U2, V, users, pos, negT, B, DIM, NNEG):
    info = plsc.get_sparse_core_info()
    NC, NS = info.num_cores, info.num_subcores
    NW = NC * NS
    bw = B // NW
    nn = (B * NNEG) // NW
    half = nn // 2
    mesh = plsc.VectorSubcoreMesh(core_axis_name="c", subcore_axis_name="s")

    @functools.partial(
        pl.kernel,
        mesh=mesh,
        out_type=[
            jax.ShapeDtypeStruct((B, DIM), jnp.float32),
            jax.ShapeDtypeStruct((B, 2 * DIM), jnp.float32),
            jax.ShapeDtypeStruct((B, DIM), jnp.float32),
            jax.ShapeDtypeStruct((B * NNEG, DIM), jnp.float32),
        ],
        scratch_types=[
            pltpu.VMEM((bw,), jnp.int32),
            pltpu.VMEM((half,), jnp.int32),
            pltpu.VMEM((bw, DIM), jnp.float32),
            pltpu.VMEM((bw, 2 * DIM), jnp.float32),
            pltpu.VMEM((half, DIM), jnp.float32),
            pltpu.SemaphoreType.DMA,
        ],
        compiler_params=pltpu.CompilerParams(use_tc_tiling_on_sc=False),
    )
    def k(ut_hbm, u2_hbm, v_hbm, users_hbm, pos_hbm, negT_hbm,
          ut_out, uu_out, i_out, j_out, idx_u, idx_n, row_v, uu_v, j_v, sem):
        wid = lax.axis_index("s") * NC + lax.axis_index("c")
        base = wid * bw
        pltpu.sync_copy(users_hbm.at[pl.ds(base, bw)], idx_u)
        c1 = pltpu.async_copy(ut_hbm.at[idx_u], row_v, sem)
        c2 = pltpu.async_copy(u2_hbm.at[idx_u], uu_v, sem)
        c1.wait()
        c2.wait()
        pltpu.sync_copy(row_v, ut_out.at[pl.ds(base, bw)])
        pltpu.sync_copy(uu_v, uu_out.at[pl.ds(base, bw)])
        pltpu.sync_copy(pos_hbm.at[pl.ds(base, bw)], idx_u)
        pltpu.async_copy(v_hbm.at[idx_u], row_v, sem).wait()
        pltpu.sync_copy(row_v, i_out.at[pl.ds(base, bw)])
        nbase = wid * nn
        for c in range(2):
            off = nbase + c * half
            pltpu.sync_copy(negT_hbm.at[pl.ds(off, half)], idx_n)
            pltpu.async_copy(v_hbm.at[idx_n], j_v, sem).wait()
            pltpu.sync_copy(j_v, j_out.at[pl.ds(off, half)])

    return k(U_true, U2, V, users, pos, negT)


def _tc_loss(ut, uu, i_rows, j3, das2, Wt, b2, da_pad, B, DIM, NNEG, DA):
    R = 2048
    NB = B // R
    NDA = da_pad.shape[0]

    def body(ut_ref, uu_ref, i_ref, j_ref, das_ref, w_ref, b_ref, dat_ref,
             out_ref):
        step = pl.program_id(0)
        s = 2.0 * ut_ref[...] + uu_ref[:, :DIM] + uu_ref[:, DIM:]
        das = jnp.minimum(jnp.maximum(das_ref[...], 0), DA)
        onehot = (das == lax.broadcasted_iota(jnp.int32, (R, NDA), 1)
                  ).astype(jnp.float32)
        u = (jnp.dot(s, w_ref[...], preferred_element_type=jnp.float32)
             + b_ref[...]
             + jnp.dot(onehot, dat_ref[...], preferred_element_type=jnp.float32))
        up = u + _EPS
        dpos = up - i_ref[...]
        dp = jnp.sqrt(jnp.sum(dpos * dpos, axis=1, keepdims=True))
        acc = jnp.zeros((), jnp.float32)
        for k in range(NNEG):
            dneg = up - j_ref[k]
            dn = jnp.sqrt(jnp.sum(dneg * dneg, axis=1, keepdims=True))
            acc = acc + jnp.sum(jnp.maximum(dp - dn + _MARGIN, 0.0))

        @pl.when(step == 0)
        def _():
            out_ref[...] = jnp.zeros_like(out_ref[...])

        out_ref[...] = out_ref[...] + acc * (1.0 / B)

    out = pl.pallas_call(
        body,
        grid=(NB,),
        in_specs=[
            pl.BlockSpec((R, DIM), lambda n: (n, 0)),
            pl.BlockSpec((R, 2 * DIM), lambda n: (n, 0)),
            pl.BlockSpec((R, DIM), lambda n: (n, 0)),
            pl.BlockSpec((NNEG, R, DIM), lambda n: (0, n, 0)),
            pl.BlockSpec((R, 1), lambda n: (n, 0)),
            pl.BlockSpec((DIM, DIM), lambda n: (0, 0)),
            pl.BlockSpec((1, DIM), lambda n: (0, 0)),
            pl.BlockSpec((NDA, DIM), lambda n: (0, 0)),
        ],
        out_specs=pl.BlockSpec((1, 1), lambda n: (0, 0)),
        out_shape=jax.ShapeDtypeStruct((1, 1), jnp.float32),
        compiler_params=pltpu.CompilerParams(
            dimension_semantics=("arbitrary",)),
    )(ut, uu, i_rows, j3, das2, Wt, b2, da_pad)
    return out[0, 0]


def kernel(phase, users, pos_job_ids, behavior_ids, das, neg_job_id_lists,
           U_true, U, V, da_tab, W, b):
    B = users.shape[0]
    DIM = U_true.shape[1]
    NNEG = neg_job_id_lists.shape[1]
    DA = da_tab.shape[0] - 1
    USER_SIZE = U.shape[0]

    U2 = U.reshape(USER_SIZE, 2 * DIM)
    negT = neg_job_id_lists.T.reshape(-1)
    ut, uu, i_rows, j_rows = _sc_gather(
        U_true, U2, V, users, pos_job_ids, negT, B, DIM, NNEG)
    j3 = j_rows.reshape(NNEG, B, DIM)
    das2 = das.reshape(B, 1)
    NDA = 128
    da_pad = jnp.pad(da_tab, ((0, NDA - (DA + 1)), (0, 0)))
    Wt = W.T
    b2 = b.reshape(1, DIM)
    return _tc_loss(ut, uu, i_rows, j3, das2, Wt, b2, da_pad,
                    B, DIM, NNEG, DA)


# speedup vs baseline: 1.5121x; 1.5121x over previous
import functools

import jax
import jax.numpy as jnp
from jax import lax
from jax.experimental import pallas as pl
from jax.experimental.pallas import tpu as pltpu
from jax.experimental.pallas import tpu_sc as plsc

_EPS = 1e-6
_MARGIN = 1.0


def _sc_gather(U_true, U2, V, users, pos, negT, B, DIM, NNEG):
    info = plsc.get_sparse_core_info()
    NC, NS = info.num_cores, info.num_subcores
    NW = NC * NS
    bw = B // NW
    nn = (B * NNEG) // NW
    half = nn // 2
    mesh = plsc.VectorSubcoreMesh(core_axis_name="c", subcore_axis_name="s")

    @functools.partial(
        pl.kernel,
        mesh=mesh,
        out_type=[
            jax.ShapeDtypeStruct((B, DIM), jnp.float32),
            jax.ShapeDtypeStruct((B, 2 * DIM), jnp.float32),
            jax.ShapeDtypeStruct((B, DIM), jnp.float32),
            jax.ShapeDtypeStruct((B * NNEG, DIM), jnp.float32),
        ],
        scratch_types=[
            pltpu.VMEM((bw,), jnp.int32),
            pltpu.VMEM((half,), jnp.int32),
            pltpu.VMEM((bw, DIM), jnp.float32),
            pltpu.VMEM((bw, 2 * DIM), jnp.float32),
            pltpu.VMEM((half, DIM), jnp.float32),
            pltpu.SemaphoreType.DMA,
        ],
        compiler_params=pltpu.CompilerParams(use_tc_tiling_on_sc=False),
    )
    def k(ut_hbm, u2_hbm, v_hbm, users_hbm, pos_hbm, negT_hbm,
          ut_out, uu_out, i_out, j_out, idx_u, idx_n, row_v, uu_v, j_v, sem):
        wid = lax.axis_index("s") * NC + lax.axis_index("c")
        base = wid * bw
        pltpu.sync_copy(users_hbm.at[pl.ds(base, bw)], idx_u)
        c1 = pltpu.async_copy(ut_hbm.at[idx_u], row_v, sem)
        c2 = pltpu.async_copy(u2_hbm.at[idx_u], uu_v, sem)
        c1.wait()
        c2.wait()
        pltpu.sync_copy(row_v, ut_out.at[pl.ds(base, bw)])
        pltpu.sync_copy(uu_v, uu_out.at[pl.ds(base, bw)])
        pltpu.sync_copy(pos_hbm.at[pl.ds(base, bw)], idx_u)
        pltpu.async_copy(v_hbm.at[idx_u], row_v, sem).wait()
        pltpu.sync_copy(row_v, i_out.at[pl.ds(base, bw)])
        nbase = wid * nn
        for c in range(2):
            off = nbase + c * half
            pltpu.sync_copy(negT_hbm.at[pl.ds(off, half)], idx_n)
            pltpu.async_copy(v_hbm.at[idx_n], j_v, sem).wait()
            pltpu.sync_copy(j_v, j_out.at[pl.ds(off, half)])

    return k(U_true, U2, V, users, pos, negT)


def _tc_loss(ut, uu, i_rows, j3, das2, Wt, b2, da_pad, B, DIM, NNEG, DA):
    R = 2048
    NB = B // R
    NDA = da_pad.shape[0]

    def body(ut_ref, uu_ref, i_ref, j_ref, das_ref, w_ref, b_ref, dat_ref,
             out_ref):
        step = pl.program_id(0)
        s = 2.0 * ut_ref[...] + uu_ref[:, :DIM] + uu_ref[:, DIM:]
        das = jnp.minimum(jnp.maximum(das_ref[...], 0), DA)
        onehot = (das == lax.broadcasted_iota(jnp.int32, (R, NDA), 1)
                  ).astype(jnp.float32)
        u = (jnp.dot(s, w_ref[...], preferred_element_type=jnp.float32)
             + b_ref[...]
             + jnp.dot(onehot, dat_ref[...], preferred_element_type=jnp.float32))
        up = u + _EPS
        dpos = up - i_ref[...]
        dp = jnp.sqrt(jnp.sum(dpos * dpos, axis=1, keepdims=True))
        acc = jnp.zeros((), jnp.float32)
        for k in range(NNEG):
            dneg = up - j_ref[k]
            dn = jnp.sqrt(jnp.sum(dneg * dneg, axis=1, keepdims=True))
            acc = acc + jnp.sum(jnp.maximum(dp - dn + _MARGIN, 0.0))

        @pl.when(step == 0)
        def _():
            out_ref[...] = jnp.zeros_like(out_ref[...])

        out_ref[...] = out_ref[...] + acc * (1.0 / B)

    out = pl.pallas_call(
        body,
        grid=(NB,),
        in_specs=[
            pl.BlockSpec((R, DIM), lambda n: (n, 0)),
            pl.BlockSpec((R, 2 * DIM), lambda n: (n, 0)),
            pl.BlockSpec((R, DIM), lambda n: (n, 0)),
            pl.BlockSpec((NNEG, R, DIM), lambda n: (0, n, 0)),
            pl.BlockSpec((R, 1), lambda n: (n, 0)),
            pl.BlockSpec((DIM, DIM), lambda n: (0, 0)),
            pl.BlockSpec((1, DIM), lambda n: (0, 0)),
            pl.BlockSpec((NDA, DIM), lambda n: (0, 0)),
        ],
        out_specs=pl.BlockSpec((1, 1), lambda n: (0, 0)),
        out_shape=jax.ShapeDtypeStruct((1, 1), jnp.float32),
        compiler_params=pltpu.CompilerParams(
            dimension_semantics=("arbitrary",)),
    )(ut, uu, i_rows, j3, das2, Wt, b2, da_pad)
    return out[0, 0]


def kernel(phase, users, pos_job_ids, behavior_ids, das, neg_job_id_lists,
           U_true, U, V, da_tab, W, b):
    B = users.shape[0]
    DIM = U_true.shape[1]
    NNEG = neg_job_id_lists.shape[1]
    DA = da_tab.shape[0] - 1
    USER_SIZE = U.shape[0]

    U_true_s = U_true[:USER_SIZE]
    U2 = U.reshape(USER_SIZE, 2 * DIM)
    negT = neg_job_id_lists.T.reshape(-1)
    ut, uu, i_rows, j_rows = _sc_gather(
        U_true_s, U2, V, users, pos_job_ids, negT, B, DIM, NNEG)
    j3 = j_rows.reshape(NNEG, B, DIM)
    das2 = das.reshape(B, 1)
    NDA = 128
    da_pad = jnp.pad(da_tab, ((0, NDA - (DA + 1)), (0, 0)))
    Wt = W.T
    b2 = b.reshape(1, DIM)
    return _tc_loss(ut, uu, i_rows, j3, das2, Wt, b2, da_pad,
                    B, DIM, NNEG, DA)
